# Initial kernel scaffold; baseline (speedup 1.0000x reference)
import jax
import jax.numpy as jnp
from jax.experimental import pallas as pl


def kernel(x, edge_index, central_mask, W_in_o, W_in_u, Wfs0, Wft0, Wg0, W_diff0, b_diff0, Wfs1, Wft1):
    raise NotImplementedError("write your pallas kernel here")


# speedup vs baseline: 61.9752x; 61.9752x over previous
import functools

import jax
import jax.numpy as jnp
from jax import lax
from jax.experimental import pallas as pl
from jax.experimental.pallas import tpu as pltpu
from jax.experimental.pallas import tpu_sc as plsc

N = 10000
E = 320000
D = 64
D2 = 2 * D
NC = 2
NS = 16
L = 16
HALF = N // NC
EPW = E // NS
BE = 4000
NB = EPW // BE
CB = 400
NG = CB // L
NCH = BE // CB
CAP = BE + L
RPS = 320
RPS_LAST = HALF - RPS * (NS - 1)
WB = 40
F32 = jnp.float32
I32 = jnp.int32

_STAGE = 3


def _leaky(a):
    return jnp.where(a >= 0.0, a, 0.1 * a)


def _tc1_body(x, cmf_r, cmf_c, wino, winu, wfs0, wft0, wg0, wdiff, bdiff,
              wfs1, wft1, xo_o, xu_o, msg0_o, s0_o, t0_o, s1_o, t1_o,
              mt0_o, c0_o, c1_o):
    xv = x[...]
    dn = (((1,), (1,)), ((), ()))
    xo = lax.dot_general(xv[:, :D], wino[...], dn, preferred_element_type=F32)
    xu = lax.dot_general(xv[:, D:], winu[...], dn, preferred_element_type=F32)
    cr = cmf_r[...]
    nc = jnp.sum(cr)
    dn2 = (((1,), (0,)), ((), ()))
    deltaX = (lax.dot_general(cr, xo, dn2, preferred_element_type=F32) / nc
              - lax.dot_general(1.0 - cr, xo, dn2, preferred_element_type=F32)
              / (N - nc))
    ad = lax.dot_general(deltaX, wdiff[...], dn,
                         preferred_element_type=F32) + bdiff[...][None, :]
    sup = (lax.dot_general(xu, wg0[...][:, :D], dn, preferred_element_type=F32)
           + lax.dot_general(ad, wg0[...][:, D:], dn,
                             preferred_element_type=F32))
    msg0 = xu - sup * cmf_c[...]
    s0 = lax.dot_general(wfs0[...], xo, dn, preferred_element_type=F32)
    t0 = lax.dot_general(wft0[...], xo, dn, preferred_element_type=F32)
    s1 = lax.dot_general(wfs1[...], xo, dn, preferred_element_type=F32)
    t1 = lax.dot_general(wft1[...], xo, dn, preferred_element_type=F32)
    c0 = _leaky(jnp.max(s0) + jnp.max(t0))
    c1 = _leaky(jnp.max(s1) + jnp.max(t1))
    xo_o[...] = xo
    xu_o[...] = xu
    msg0_o[...] = jnp.concatenate([msg0, jnp.zeros((N, D), F32)], axis=1)
    s0_o[...] = s0
    t0_o[...] = t0
    s1_o[...] = s1
    t1_o[...] = t1
    mt0_o[...] = 1.0 - cr
    c0_o[...] = jnp.full((16,), c0, F32)
    c1_o[...] = jnp.full((16,), c1, F32)


_tc1 = pl.pallas_call(
    _tc1_body,
    out_shape=(
        jax.ShapeDtypeStruct((N, D), F32),
        jax.ShapeDtypeStruct((N, D), F32),
        jax.ShapeDtypeStruct((N, D2), F32),
        jax.ShapeDtypeStruct((1, N), F32),
        jax.ShapeDtypeStruct((1, N), F32),
        jax.ShapeDtypeStruct((1, N), F32),
        jax.ShapeDtypeStruct((1, N), F32),
        jax.ShapeDtypeStruct((1, N), F32),
        jax.ShapeDtypeStruct((16,), F32),
        jax.ShapeDtypeStruct((16,), F32),
    ),
)


def _sc_body(esrc, edst, s_t, t_t, ms_t, mt_t, cvec, msg, den_o, out_o,
             s_v, t_v, ms_v, mt_v, cvec_v, srcb, dstb, srcC, dstC, exC,
             rowbuf, rowbuf2, zrow, wbuf, den_t, out_acc, den_acc,
             sem, sem2, sem3):
    c = lax.axis_index("c")
    s = lax.axis_index("s")
    lo = c * HALF
    r0 = pl.multiple_of(s * RPS, 8)
    z16 = jnp.zeros((L,), F32)
    z16i = jnp.zeros((L,), I32)

    if _STAGE <= -3:
        return
    pltpu.sync_copy(s_t, s_v)
    pltpu.sync_copy(t_t, t_v)
    pltpu.sync_copy(ms_t, ms_v)
    pltpu.sync_copy(mt_t, mt_v)
    pltpu.sync_copy(cvec, cvec_v)
    if _STAGE <= -2:
        return
    for j in range(8):
        for k in range(D2 // L):
            zrow[j, pl.ds(k * L, L)] = z16
            rowbuf2[j, pl.ds(k * L, L)] = z16
            rowbuf2[j + 8, pl.ds(k * L, L)] = z16

    def _zv(i, carry):
        den_t[pl.ds(i * L, L)] = z16
        return carry
    lax.fori_loop(0, 640 // L, _zv, 0)

    def _stage(sz):
        pltpu.sync_copy(den_t.at[pl.ds(0, sz)], den_acc.at[pl.ds(r0, sz)])

        def _zo(i, carry):
            pltpu.sync_copy(zrow, out_acc.at[pl.ds(r0 + i * 8, 8)])
            return carry
        lax.fori_loop(0, sz // 8, _zo, 0)

    @pl.when(s < NS - 1)
    def _():
        _stage(RPS)

    @pl.when(s == NS - 1)
    def _():
        _stage(RPS_LAST)

    plsc.subcore_barrier()
    if _STAGE <= -1:
        return

    Cv = plsc.load_gather(cvec_v, [z16i])

    if _STAGE >= 1:
        for b in range(NB):
            def _chunk(ch, cnt, b=b):
                base = pl.multiple_of(s * EPW + b * BE + ch * CB, 8)
                pltpu.sync_copy(esrc.at[pl.ds(base, CB)], srcb)
                pltpu.sync_copy(edst.at[pl.ds(base, CB)], dstb)

                def _grp(g, cnt):
                    sv = srcb[pl.ds(g * L, L)]
                    dv = dstb[pl.ds(g * L, L)]
                    dl = dv - lo
                    e = (plsc.load_gather(ms_v, [sv])
                         * plsc.load_gather(mt_v, [dv]))
                    a = (plsc.load_gather(s_v, [sv])
                         + plsc.load_gather(t_v, [dv]))
                    ex = e * jnp.exp(_leaky(a) - Cv)
                    msk = (e > 0.0) & (dl >= 0) & (dl < HALF)
                    plsc.store_compressed(srcC.at[pl.ds(cnt, L)], sv, mask=msk)
                    plsc.store_compressed(dstC.at[pl.ds(cnt, L)], dl, mask=msk)
                    plsc.store_compressed(exC.at[pl.ds(cnt, L)], ex, mask=msk)
                    return cnt + jnp.sum(msk.astype(I32))
                return lax.fori_loop(0, NG, _grp, cnt)

            cnt = lax.fori_loop(0, NCH, _chunk, jnp.int32(0))
            srcC[pl.ds(cnt, L)] = z16i
            dstC[pl.ds(cnt, L)] = z16i
            exC[pl.ds(cnt, L)] = z16
            ngrp = (cnt + (L - 1)) // L

            if _STAGE >= 2:
                def _rg(g, carry):
                    o = pl.multiple_of(g * L, 8)
                    sv = srcC[pl.ds(o, L)]
                    dl = dstC[pl.ds(o, L)]
                    pltpu.async_copy(msg.at[sv], rowbuf, sem).wait()
                    for j in range(L):
                        exj = plsc.load_gather(
                            exC, [jnp.full((L,), o + j, I32)])
                        for k in range(D // L):
                            rowbuf2[j, pl.ds(k * L, L)] = (
                                rowbuf[j, pl.ds(k * L, L)] * exj)
                    if _STAGE >= 3:
                        pltpu.async_copy(rowbuf2, out_acc.at[dl], sem2,
                                         add=True).wait()
                        pltpu.async_copy(exC.at[pl.ds(o, L)],
                                         den_acc.at[dl], sem3,
                                         add=True).wait()
                    return carry
                lax.fori_loop(0, ngrp, _rg, 0)

    plsc.subcore_barrier()

    def _wb(sz):
        pltpu.sync_copy(den_acc.at[pl.ds(r0, sz)], den_t.at[pl.ds(0, sz)])
        pltpu.sync_copy(den_t.at[pl.ds(0, sz)],
                        den_o.at[pl.ds(pl.multiple_of(lo + r0, 8), sz)])

        def _wo(i, carry):
            ro = r0 + i * WB
            pltpu.sync_copy(out_acc.at[pl.ds(ro, WB)], wbuf)
            pltpu.sync_copy(wbuf, out_o.at[pl.ds(lo + ro, WB)])
            return carry
        lax.fori_loop(0, sz // WB, _wo, 0)

    @pl.when(s < NS - 1)
    def _():
        _wb(RPS)

    @pl.when(s == NS - 1)
    def _():
        _wb(RPS_LAST)


@functools.lru_cache(maxsize=1)
def _make_sc_layer():
    return pl.kernel(
        _sc_body,
        out_type=(
            jax.ShapeDtypeStruct((N,), F32),
            jax.ShapeDtypeStruct((N, D2), F32),
        ),
        mesh=plsc.VectorSubcoreMesh(core_axis_name="c", subcore_axis_name="s",
                                    num_cores=NC, num_subcores=NS),
        compiler_params=pltpu.CompilerParams(needs_layout_passes=False),
        scratch_types=(
            pltpu.VMEM((N,), F32),
            pltpu.VMEM((N,), F32),
            pltpu.VMEM((N,), F32),
            pltpu.VMEM((N,), F32),
            pltpu.VMEM((16,), F32),
            pltpu.VMEM((CB,), I32),
            pltpu.VMEM((CB,), I32),
            pltpu.VMEM((CAP,), I32),
            pltpu.VMEM((CAP,), I32),
            pltpu.VMEM((CAP,), F32),
            pltpu.VMEM((L, D2), F32),
            pltpu.VMEM((L, D2), F32),
            pltpu.VMEM((8, D2), F32),
            pltpu.VMEM((WB, D2), F32),
            pltpu.VMEM((640,), F32),
            pltpu.VMEM_SHARED((HALF, D2), F32),
            pltpu.VMEM_SHARED((HALF + 8,), F32),
            pltpu.SemaphoreType.DMA,
            pltpu.SemaphoreType.DMA,
            pltpu.SemaphoreType.DMA,
        ),
    )


def _tc2_body(den_r, outU, cmf_r, msg1_o, ms1_o, mt1_o):
    den = den_r[...]
    den1d = jnp.sum(den, axis=0)
    rec = 1.0 / (den1d + 1e-16)
    msg1_o[...] = outU[...] * rec[:, None]
    r = (den > 0.0).astype(F32)
    ms1_o[...] = r
    mt1_o[...] = (1.0 - cmf_r[...]) * (1.0 - r)


_tc2 = pl.pallas_call(
    _tc2_body,
    out_shape=(
        jax.ShapeDtypeStruct((N, D2), F32),
        jax.ShapeDtypeStruct((1, N), F32),
        jax.ShapeDtypeStruct((1, N), F32),
    ),
)


def _tc3_body(den_r, outU, xo, xu, cmf_c, out_o):
    den1d = jnp.sum(den_r[...], axis=0)
    rec = 1.0 / (den1d + 1e-16)
    xuh = outU[...][:, :D] * rec[:, None]
    cc = cmf_c[...]
    out_o[...] = jnp.concatenate(
        [xo[...], xu[...] * cc + xuh * (1.0 - cc)], axis=1)


_tc3 = pl.pallas_call(
    _tc3_body,
    out_shape=jax.ShapeDtypeStruct((N, D2), F32),
)


def kernel(x, edge_index, central_mask, W_in_o, W_in_u, Wfs0, Wft0, Wg0,
           W_diff0, b_diff0, Wfs1, Wft1):
    cmf_r = central_mask.astype(F32).reshape(1, N)
    cmf_c = central_mask.astype(F32).reshape(N, 1)
    (xo, xu, msg0, s0, t0, s1, t1, mt0, c0, c1) = _tc1(
        x, cmf_r, cmf_c, W_in_o, W_in_u, Wfs0, Wft0, Wg0, W_diff0, b_diff0,
        Wfs1, Wft1)
    esrc = edge_index[0]
    edst = edge_index[1]
    _sc_layer = _make_sc_layer()
    den0, out0 = _sc_layer(esrc, edst, s0.reshape(N), t0.reshape(N),
                           cmf_r.reshape(N), mt0.reshape(N), c0, msg0)
    msg1, ms1, mt1 = _tc2(den0.reshape(1, N), out0, cmf_r)
    den1, out1 = _sc_layer(esrc, edst, s1.reshape(N), t1.reshape(N),
                           ms1.reshape(N), mt1.reshape(N), c1, msg1)
    return _tc3(den1.reshape(1, N), out1, xo, xu, cmf_c)


# speedup vs baseline: 62.0207x; 1.0007x over previous
import functools

import jax
import jax.numpy as jnp
from jax import lax
from jax.experimental import pallas as pl
from jax.experimental.pallas import tpu as pltpu
from jax.experimental.pallas import tpu_sc as plsc

N = 10000
E = 320000
D = 64
D2 = 2 * D
NC = 2
NS = 16
L = 16
HALF = N // NC
EPW = E // NS
BE = 4000
NB = EPW // BE
CB = 400
NG = CB // L
NCH = BE // CB
CAP = BE + L
RPS = 320
RPS_LAST = HALF - RPS * (NS - 1)
WB = 40
F32 = jnp.float32
I32 = jnp.int32


def _leaky(a):
    return jnp.where(a >= 0.0, a, 0.1 * a)


def _tc1_body(x, cmf_r, cmf_c, wino, winu, wfs0, wft0, wg0, wdiff, bdiff,
              wfs1, wft1, xo_o, xu_o, msg0_o, s0_o, t0_o, s1_o, t1_o,
              mt0_o, c0_o, c1_o):
    xv = x[...]
    dn = (((1,), (1,)), ((), ()))
    xo = lax.dot_general(xv[:, :D], wino[...], dn, preferred_element_type=F32)
    xu = lax.dot_general(xv[:, D:], winu[...], dn, preferred_element_type=F32)
    cr = cmf_r[...]
    nc = jnp.sum(cr)
    dn2 = (((1,), (0,)), ((), ()))
    deltaX = (lax.dot_general(cr, xo, dn2, preferred_element_type=F32) / nc
              - lax.dot_general(1.0 - cr, xo, dn2, preferred_element_type=F32)
              / (N - nc))
    ad = lax.dot_general(deltaX, wdiff[...], dn,
                         preferred_element_type=F32) + bdiff[...][None, :]
    sup = (lax.dot_general(xu, wg0[...][:, :D], dn, preferred_element_type=F32)
           + lax.dot_general(ad, wg0[...][:, D:], dn,
                             preferred_element_type=F32))
    msg0 = xu - sup * cmf_c[...]
    s0 = lax.dot_general(wfs0[...], xo, dn, preferred_element_type=F32)
    t0 = lax.dot_general(wft0[...], xo, dn, preferred_element_type=F32)
    s1 = lax.dot_general(wfs1[...], xo, dn, preferred_element_type=F32)
    t1 = lax.dot_general(wft1[...], xo, dn, preferred_element_type=F32)
    c0 = _leaky(jnp.max(s0) + jnp.max(t0))
    c1 = _leaky(jnp.max(s1) + jnp.max(t1))
    xo_o[...] = xo
    xu_o[...] = xu
    msg0_o[...] = jnp.concatenate([msg0, jnp.zeros((N, D), F32)], axis=1)
    s0_o[...] = s0
    t0_o[...] = t0
    s1_o[...] = s1
    t1_o[...] = t1
    mt0_o[...] = 1.0 - cr
    c0_o[...] = jnp.full((16,), c0, F32)
    c1_o[...] = jnp.full((16,), c1, F32)


_tc1 = pl.pallas_call(
    _tc1_body,
    out_shape=(
        jax.ShapeDtypeStruct((N, D), F32),
        jax.ShapeDtypeStruct((N, D), F32),
        jax.ShapeDtypeStruct((N, D2), F32),
        jax.ShapeDtypeStruct((1, N), F32),
        jax.ShapeDtypeStruct((1, N), F32),
        jax.ShapeDtypeStruct((1, N), F32),
        jax.ShapeDtypeStruct((1, N), F32),
        jax.ShapeDtypeStruct((1, N), F32),
        jax.ShapeDtypeStruct((16,), F32),
        jax.ShapeDtypeStruct((16,), F32),
    ),
)


def _sc_body(esrc, edst, s_t, t_t, ms_t, mt_t, cvec, msg, den_o, out_o,
             s_v, t_v, ms_v, mt_v, cvec_v, srcb, dstb, srcC, dstC, exC,
             rowbuf, rowbuf2, zrow, wbuf, den_t, out_acc, den_acc,
             sem, sem2, sem3):
    c = lax.axis_index("c")
    s = lax.axis_index("s")
    lo = c * HALF
    r0 = pl.multiple_of(s * RPS, 8)
    z16 = jnp.zeros((L,), F32)
    z16i = jnp.zeros((L,), I32)

    pltpu.sync_copy(s_t, s_v)
    pltpu.sync_copy(t_t, t_v)
    pltpu.sync_copy(ms_t, ms_v)
    pltpu.sync_copy(mt_t, mt_v)
    pltpu.sync_copy(cvec, cvec_v)
    for j in range(8):
        for k in range(D2 // L):
            zrow[j, pl.ds(k * L, L)] = z16
            rowbuf2[j, pl.ds(k * L, L)] = z16
            rowbuf2[j + 8, pl.ds(k * L, L)] = z16

    def _zv(i, carry):
        den_t[pl.ds(i * L, L)] = z16
        return carry
    lax.fori_loop(0, 640 // L, _zv, 0)

    def _stage(sz):
        pltpu.sync_copy(den_t.at[pl.ds(0, sz)], den_acc.at[pl.ds(r0, sz)])

        def _zo(i, carry):
            pltpu.sync_copy(zrow, out_acc.at[pl.ds(r0 + i * 8, 8)])
            return carry
        lax.fori_loop(0, sz // 8, _zo, 0)

    @pl.when(s < NS - 1)
    def _():
        _stage(RPS)

    @pl.when(s == NS - 1)
    def _():
        _stage(RPS_LAST)

    plsc.subcore_barrier()

    Cv = plsc.load_gather(cvec_v, [z16i])

    for b in range(NB):
        def _chunk(ch, cnt, b=b):
            base = pl.multiple_of(s * EPW + b * BE + ch * CB, 8)
            pltpu.sync_copy(esrc.at[pl.ds(base, CB)], srcb)
            pltpu.sync_copy(edst.at[pl.ds(base, CB)], dstb)

            def _grp(g, cnt):
                sv = srcb[pl.ds(g * L, L)]
                dv = dstb[pl.ds(g * L, L)]
                dl = dv - lo
                e = (plsc.load_gather(ms_v, [sv])
                     * plsc.load_gather(mt_v, [dv]))
                a = (plsc.load_gather(s_v, [sv])
                     + plsc.load_gather(t_v, [dv]))
                ex = e * jnp.exp(_leaky(a) - Cv)
                msk = (e > 0.0) & (dl >= 0) & (dl < HALF)
                plsc.store_compressed(srcC.at[pl.ds(cnt, L)], sv, mask=msk)
                plsc.store_compressed(dstC.at[pl.ds(cnt, L)], dl, mask=msk)
                plsc.store_compressed(exC.at[pl.ds(cnt, L)], ex, mask=msk)
                return cnt + jnp.sum(msk.astype(I32))
            return lax.fori_loop(0, NG, _grp, cnt)

        cnt = lax.fori_loop(0, NCH, _chunk, jnp.int32(0))
        srcC[pl.ds(cnt, L)] = z16i
        dstC[pl.ds(cnt, L)] = z16i
        exC[pl.ds(cnt, L)] = z16
        ngrp = (cnt + (L - 1)) // L

        def _rg(g, carry):
            o = pl.multiple_of(g * L, 8)
            sv = srcC[pl.ds(o, L)]
            dl = dstC[pl.ds(o, L)]
            pltpu.async_copy(msg.at[sv], rowbuf, sem).wait()
            for j in range(L):
                exj = plsc.load_gather(exC, [jnp.full((L,), o + j, I32)])
                for k in range(D // L):
                    rowbuf2[j, pl.ds(k * L, L)] = (
                        rowbuf[j, pl.ds(k * L, L)] * exj)
            pltpu.async_copy(rowbuf2, out_acc.at[dl], sem2, add=True).wait()
            pltpu.async_copy(exC.at[pl.ds(o, L)], den_acc.at[dl], sem3,
                             add=True).wait()
            return carry
        lax.fori_loop(0, ngrp, _rg, 0)

    plsc.subcore_barrier()

    def _wb(sz):
        pltpu.sync_copy(den_acc.at[pl.ds(r0, sz)], den_t.at[pl.ds(0, sz)])
        pltpu.sync_copy(den_t.at[pl.ds(0, sz)],
                        den_o.at[pl.ds(pl.multiple_of(lo + r0, 8), sz)])

        def _wo(i, carry):
            ro = r0 + i * WB
            pltpu.sync_copy(out_acc.at[pl.ds(ro, WB)], wbuf)
            pltpu.sync_copy(wbuf, out_o.at[pl.ds(lo + ro, WB)])
            return carry
        lax.fori_loop(0, sz // WB, _wo, 0)

    @pl.when(s < NS - 1)
    def _():
        _wb(RPS)

    @pl.when(s == NS - 1)
    def _():
        _wb(RPS_LAST)


@functools.lru_cache(maxsize=1)
def _make_sc_layer():
    return pl.kernel(
        _sc_body,
        out_type=(
            jax.ShapeDtypeStruct((N,), F32),
            jax.ShapeDtypeStruct((N, D2), F32),
        ),
        mesh=plsc.VectorSubcoreMesh(core_axis_name="c", subcore_axis_name="s",
                                    num_cores=NC, num_subcores=NS),
        compiler_params=pltpu.CompilerParams(needs_layout_passes=False),
        scratch_types=(
            pltpu.VMEM((N,), F32),
            pltpu.VMEM((N,), F32),
            pltpu.VMEM((N,), F32),
            pltpu.VMEM((N,), F32),
            pltpu.VMEM((16,), F32),
            pltpu.VMEM((CB,), I32),
            pltpu.VMEM((CB,), I32),
            pltpu.VMEM((CAP,), I32),
            pltpu.VMEM((CAP,), I32),
            pltpu.VMEM((CAP,), F32),
            pltpu.VMEM((L, D2), F32),
            pltpu.VMEM((L, D2), F32),
            pltpu.VMEM((8, D2), F32),
            pltpu.VMEM((WB, D2), F32),
            pltpu.VMEM((640,), F32),
            pltpu.VMEM_SHARED((HALF, D2), F32),
            pltpu.VMEM_SHARED((HALF + 8,), F32),
            pltpu.SemaphoreType.DMA,
            pltpu.SemaphoreType.DMA,
            pltpu.SemaphoreType.DMA,
        ),
    )


def _tc2_body(den_r, outU, cmf_r, msg1_o, ms1_o, mt1_o):
    den = den_r[...]
    den1d = jnp.sum(den, axis=0)
    rec = 1.0 / (den1d + 1e-16)
    msg1_o[...] = outU[...] * rec[:, None]
    r = (den > 0.0).astype(F32)
    ms1_o[...] = r
    mt1_o[...] = (1.0 - cmf_r[...]) * (1.0 - r)


_tc2 = pl.pallas_call(
    _tc2_body,
    out_shape=(
        jax.ShapeDtypeStruct((N, D2), F32),
        jax.ShapeDtypeStruct((1, N), F32),
        jax.ShapeDtypeStruct((1, N), F32),
    ),
)


def _tc3_body(den_r, outU, xo, xu, cmf_c, out_o):
    den1d = jnp.sum(den_r[...], axis=0)
    rec = 1.0 / (den1d + 1e-16)
    xuh = outU[...][:, :D] * rec[:, None]
    cc = cmf_c[...]
    out_o[...] = jnp.concatenate(
        [xo[...], xu[...] * cc + xuh * (1.0 - cc)], axis=1)


_tc3 = pl.pallas_call(
    _tc3_body,
    out_shape=jax.ShapeDtypeStruct((N, D2), F32),
)


def kernel(x, edge_index, central_mask, W_in_o, W_in_u, Wfs0, Wft0, Wg0,
           W_diff0, b_diff0, Wfs1, Wft1):
    cmf_r = central_mask.astype(F32).reshape(1, N)
    cmf_c = central_mask.astype(F32).reshape(N, 1)
    (xo, xu, msg0, s0, t0, s1, t1, mt0, c0, c1) = _tc1(
        x, cmf_r, cmf_c, W_in_o, W_in_u, Wfs0, Wft0, Wg0, W_diff0, b_diff0,
        Wfs1, Wft1)
    esrc = edge_index[0]
    edst = edge_index[1]
    _sc_layer = _make_sc_layer()
    den0, out0 = _sc_layer(esrc, edst, s0.reshape(N), t0.reshape(N),
                           cmf_r.reshape(N), mt0.reshape(N), c0, msg0)
    msg1, ms1, mt1 = _tc2(den0.reshape(1, N), out0, cmf_r)
    den1, out1 = _sc_layer(esrc, edst, s1.reshape(N), t1.reshape(N),
                           ms1.reshape(N), mt1.reshape(N), c1, msg1)
    return _tc3(den1.reshape(1, N), out1, xo, xu, cmf_c)


# speedup vs baseline: 68.7258x; 1.1081x over previous
import functools

import jax
import jax.numpy as jnp
from jax import lax
from jax.experimental import pallas as pl
from jax.experimental.pallas import tpu as pltpu
from jax.experimental.pallas import tpu_sc as plsc

N = 10000
E = 320000
D = 64
D2 = 2 * D
NC = 2
NS = 16
L = 16
HALF = N // NC
EPW = E // NS
BE = 4000
NB = EPW // BE
CB = 400
NG = CB // L
NCH = BE // CB
CAP = BE + 2 * L
RPS = 320
RPS_LAST = HALF - RPS * (NS - 1)
WB = 40
F32 = jnp.float32
I32 = jnp.int32


def _leaky(a):
    return jnp.where(a >= 0.0, a, 0.1 * a)


def _tc1_body(x, cmf_r, cmf_c, wino, winu, wfs0, wft0, wg0, wdiff, bdiff,
              wfs1, wft1, xo_o, xu_o, msg0_o, s0_o, t0_o, s1_o, t1_o,
              mt0_o, c0_o, c1_o):
    xv = x[...]
    dn = (((1,), (1,)), ((), ()))
    xo = lax.dot_general(xv[:, :D], wino[...], dn, preferred_element_type=F32)
    xu = lax.dot_general(xv[:, D:], winu[...], dn, preferred_element_type=F32)
    cr = cmf_r[...]
    nc = jnp.sum(cr)
    dn2 = (((1,), (0,)), ((), ()))
    deltaX = (lax.dot_general(cr, xo, dn2, preferred_element_type=F32) / nc
              - lax.dot_general(1.0 - cr, xo, dn2, preferred_element_type=F32)
              / (N - nc))
    ad = lax.dot_general(deltaX, wdiff[...], dn,
                         preferred_element_type=F32) + bdiff[...][None, :]
    sup = (lax.dot_general(xu, wg0[...][:, :D], dn, preferred_element_type=F32)
           + lax.dot_general(ad, wg0[...][:, D:], dn,
                             preferred_element_type=F32))
    msg0 = xu - sup * cmf_c[...]
    s0 = lax.dot_general(wfs0[...], xo, dn, preferred_element_type=F32)
    t0 = lax.dot_general(wft0[...], xo, dn, preferred_element_type=F32)
    s1 = lax.dot_general(wfs1[...], xo, dn, preferred_element_type=F32)
    t1 = lax.dot_general(wft1[...], xo, dn, preferred_element_type=F32)
    c0 = _leaky(jnp.max(s0) + jnp.max(t0))
    c1 = _leaky(jnp.max(s1) + jnp.max(t1))
    xo_o[...] = xo
    xu_o[...] = xu
    msg0_o[...] = jnp.concatenate([msg0, jnp.zeros((N, D), F32)], axis=1)
    s0_o[...] = s0
    t0_o[...] = t0
    s1_o[...] = s1
    t1_o[...] = t1
    mt0_o[...] = 1.0 - cr
    c0_o[...] = jnp.full((16,), c0, F32)
    c1_o[...] = jnp.full((16,), c1, F32)


_tc1 = pl.pallas_call(
    _tc1_body,
    out_shape=(
        jax.ShapeDtypeStruct((N, D), F32),
        jax.ShapeDtypeStruct((N, D), F32),
        jax.ShapeDtypeStruct((N, D2), F32),
        jax.ShapeDtypeStruct((1, N), F32),
        jax.ShapeDtypeStruct((1, N), F32),
        jax.ShapeDtypeStruct((1, N), F32),
        jax.ShapeDtypeStruct((1, N), F32),
        jax.ShapeDtypeStruct((1, N), F32),
        jax.ShapeDtypeStruct((16,), F32),
        jax.ShapeDtypeStruct((16,), F32),
    ),
)


def _sc_body(esrc, edst, s_t, t_t, ms_t, mt_t, cvec, msg, den_o, out_o,
             s_v, t_v, ms_v, mt_v, cvec_v, srcb, dstb, srcC, dstC, exC,
             rowbufA, rowbufB, rowbuf2A, rowbuf2B, zrow, wbuf, den_t,
             out_acc, den_acc, sem, semB, sem2, sem3):
    c = lax.axis_index("c")
    s = lax.axis_index("s")
    lo = c * HALF
    r0 = pl.multiple_of(s * RPS, 8)
    z16 = jnp.zeros((L,), F32)
    z16i = jnp.zeros((L,), I32)

    pltpu.sync_copy(s_t, s_v)
    pltpu.sync_copy(t_t, t_v)
    pltpu.sync_copy(ms_t, ms_v)
    pltpu.sync_copy(mt_t, mt_v)
    pltpu.sync_copy(cvec, cvec_v)
    for j in range(8):
        for k in range(D2 // L):
            zrow[j, pl.ds(k * L, L)] = z16
            rowbuf2A[j, pl.ds(k * L, L)] = z16
            rowbuf2A[j + 8, pl.ds(k * L, L)] = z16
            rowbuf2B[j, pl.ds(k * L, L)] = z16
            rowbuf2B[j + 8, pl.ds(k * L, L)] = z16

    def _zv(i, carry):
        den_t[pl.ds(i * L, L)] = z16
        return carry
    lax.fori_loop(0, 640 // L, _zv, 0)

    def _stage(sz):
        pltpu.sync_copy(den_t.at[pl.ds(0, sz)], den_acc.at[pl.ds(r0, sz)])

        def _zo(i, carry):
            pltpu.sync_copy(zrow, out_acc.at[pl.ds(r0 + i * 8, 8)])
            return carry
        lax.fori_loop(0, sz // 8, _zo, 0)

    @pl.when(s < NS - 1)
    def _():
        _stage(RPS)

    @pl.when(s == NS - 1)
    def _():
        _stage(RPS_LAST)

    plsc.subcore_barrier()

    Cv = plsc.load_gather(cvec_v, [z16i])

    for b in range(NB):
        def _chunk(ch, cnt, b=b):
            base = pl.multiple_of(s * EPW + b * BE + ch * CB, 8)
            pltpu.sync_copy(esrc.at[pl.ds(base, CB)], srcb)
            pltpu.sync_copy(edst.at[pl.ds(base, CB)], dstb)

            def _grp(g, cnt):
                sv = srcb[pl.ds(g * L, L)]
                dv = dstb[pl.ds(g * L, L)]
                dl = dv - lo
                e = (plsc.load_gather(ms_v, [sv])
                     * plsc.load_gather(mt_v, [dv]))
                a = (plsc.load_gather(s_v, [sv])
                     + plsc.load_gather(t_v, [dv]))
                ex = e * jnp.exp(_leaky(a) - Cv)
                msk = (e > 0.0) & (dl >= 0) & (dl < HALF)
                plsc.store_compressed(srcC.at[pl.ds(cnt, L)], sv, mask=msk)
                plsc.store_compressed(dstC.at[pl.ds(cnt, L)], dl, mask=msk)
                plsc.store_compressed(exC.at[pl.ds(cnt, L)], ex, mask=msk)
                return cnt + jnp.sum(msk.astype(I32))
            return lax.fori_loop(0, NG, _grp, cnt)

        cnt = lax.fori_loop(0, NCH, _chunk, jnp.int32(0))
        srcC[pl.ds(cnt, L)] = z16i
        dstC[pl.ds(cnt, L)] = z16i
        exC[pl.ds(cnt, L)] = z16
        srcC[pl.ds(cnt + L, L)] = z16i
        dstC[pl.ds(cnt + L, L)] = z16i
        exC[pl.ds(cnt + L, L)] = z16
        npair = (cnt + (2 * L - 1)) // (2 * L)

        def _scale(buf, buf2, o):
            for j in range(L):
                exj = plsc.load_gather(exC, [jnp.full((L,), o + j, I32)])
                for k in range(D // L):
                    buf2[j, pl.ds(k * L, L)] = buf[j, pl.ds(k * L, L)] * exj

        def _pair(p, carry):
            o0 = pl.multiple_of(p * 2 * L, 8)
            o1 = pl.multiple_of(p * 2 * L + L, 8)
            sv0 = srcC[pl.ds(o0, L)]
            sv1 = srcC[pl.ds(o1, L)]
            dl0 = dstC[pl.ds(o0, L)]
            dl1 = dstC[pl.ds(o1, L)]
            g0 = pltpu.async_copy(msg.at[sv0], rowbufA, sem)
            g1 = pltpu.async_copy(msg.at[sv1], rowbufB, semB)
            g0.wait()
            _scale(rowbufA, rowbuf2A, o0)
            sA = pltpu.async_copy(rowbuf2A, out_acc.at[dl0], sem2, add=True)
            dA = pltpu.async_copy(exC.at[pl.ds(o0, L)], den_acc.at[dl0],
                                  sem3, add=True)
            g1.wait()
            _scale(rowbufB, rowbuf2B, o1)
            sB = pltpu.async_copy(rowbuf2B, out_acc.at[dl1], sem2, add=True)
            dB = pltpu.async_copy(exC.at[pl.ds(o1, L)], den_acc.at[dl1],
                                  sem3, add=True)
            sA.wait()
            dA.wait()
            sB.wait()
            dB.wait()
            return carry
        lax.fori_loop(0, npair, _pair, 0)

    plsc.subcore_barrier()

    def _wb(sz):
        pltpu.sync_copy(den_acc.at[pl.ds(r0, sz)], den_t.at[pl.ds(0, sz)])
        pltpu.sync_copy(den_t.at[pl.ds(0, sz)],
                        den_o.at[pl.ds(pl.multiple_of(lo + r0, 8), sz)])

        def _wo(i, carry):
            ro = r0 + i * WB
            pltpu.sync_copy(out_acc.at[pl.ds(ro, WB)], wbuf)
            pltpu.sync_copy(wbuf, out_o.at[pl.ds(lo + ro, WB)])
            return carry
        lax.fori_loop(0, sz // WB, _wo, 0)

    @pl.when(s < NS - 1)
    def _():
        _wb(RPS)

    @pl.when(s == NS - 1)
    def _():
        _wb(RPS_LAST)


@functools.lru_cache(maxsize=1)
def _make_sc_layer():
    return pl.kernel(
        _sc_body,
        out_type=(
            jax.ShapeDtypeStruct((N,), F32),
            jax.ShapeDtypeStruct((N, D2), F32),
        ),
        mesh=plsc.VectorSubcoreMesh(core_axis_name="c", subcore_axis_name="s",
                                    num_cores=NC, num_subcores=NS),
        compiler_params=pltpu.CompilerParams(needs_layout_passes=False),
        scratch_types=(
            pltpu.VMEM((N,), F32),
            pltpu.VMEM((N,), F32),
            pltpu.VMEM((N,), F32),
            pltpu.VMEM((N,), F32),
            pltpu.VMEM((16,), F32),
            pltpu.VMEM((CB,), I32),
            pltpu.VMEM((CB,), I32),
            pltpu.VMEM((CAP,), I32),
            pltpu.VMEM((CAP,), I32),
            pltpu.VMEM((CAP,), F32),
            pltpu.VMEM((L, D2), F32),
            pltpu.VMEM((L, D2), F32),
            pltpu.VMEM((L, D2), F32),
            pltpu.VMEM((L, D2), F32),
            pltpu.VMEM((8, D2), F32),
            pltpu.VMEM((WB, D2), F32),
            pltpu.VMEM((640,), F32),
            pltpu.VMEM_SHARED((HALF, D2), F32),
            pltpu.VMEM_SHARED((HALF + 8,), F32),
            pltpu.SemaphoreType.DMA,
            pltpu.SemaphoreType.DMA,
            pltpu.SemaphoreType.DMA,
            pltpu.SemaphoreType.DMA,
        ),
    )


def _tc2_body(den_r, outU, cmf_r, msg1_o, ms1_o, mt1_o):
    den = den_r[...]
    den1d = jnp.sum(den, axis=0)
    rec = 1.0 / (den1d + 1e-16)
    msg1_o[...] = outU[...] * rec[:, None]
    r = (den > 0.0).astype(F32)
    ms1_o[...] = r
    mt1_o[...] = (1.0 - cmf_r[...]) * (1.0 - r)


_tc2 = pl.pallas_call(
    _tc2_body,
    out_shape=(
        jax.ShapeDtypeStruct((N, D2), F32),
        jax.ShapeDtypeStruct((1, N), F32),
        jax.ShapeDtypeStruct((1, N), F32),
    ),
)


def _tc3_body(den_r, outU, xo, xu, cmf_c, out_o):
    den1d = jnp.sum(den_r[...], axis=0)
    rec = 1.0 / (den1d + 1e-16)
    xuh = outU[...][:, :D] * rec[:, None]
    cc = cmf_c[...]
    out_o[...] = jnp.concatenate(
        [xo[...], xu[...] * cc + xuh * (1.0 - cc)], axis=1)


_tc3 = pl.pallas_call(
    _tc3_body,
    out_shape=jax.ShapeDtypeStruct((N, D2), F32),
)


def kernel(x, edge_index, central_mask, W_in_o, W_in_u, Wfs0, Wft0, Wg0,
           W_diff0, b_diff0, Wfs1, Wft1):
    cmf_r = central_mask.astype(F32).reshape(1, N)
    cmf_c = central_mask.astype(F32).reshape(N, 1)
    (xo, xu, msg0, s0, t0, s1, t1, mt0, c0, c1) = _tc1(
        x, cmf_r, cmf_c, W_in_o, W_in_u, Wfs0, Wft0, Wg0, W_diff0, b_diff0,
        Wfs1, Wft1)
    esrc = edge_index[0]
    edst = edge_index[1]
    _sc_layer = _make_sc_layer()
    den0, out0 = _sc_layer(esrc, edst, s0.reshape(N), t0.reshape(N),
                           cmf_r.reshape(N), mt0.reshape(N), c0, msg0)
    msg1, ms1, mt1 = _tc2(den0.reshape(1, N), out0, cmf_r)
    den1, out1 = _sc_layer(esrc, edst, s1.reshape(N), t1.reshape(N),
                           ms1.reshape(N), mt1.reshape(N), c1, msg1)
    return _tc3(den1.reshape(1, N), out1, xo, xu, cmf_c)
